# Initial kernel scaffold; baseline (speedup 1.0000x reference)
import jax
import jax.numpy as jnp
from jax.experimental import pallas as pl


def kernel(skill_embed, adj_list, edge_attr, W1, b1, W2, b2, W3, b3):
    raise NotImplementedError("write your pallas kernel here")


# speedup vs baseline: 7.8953x; 7.8953x over previous
import functools

import jax
import jax.numpy as jnp
from jax import lax
from jax.experimental import pallas as pl
from jax.experimental.pallas import tpu as pltpu
from jax.experimental.pallas import tpu_sc as plsc

N = 10000
E = 320000
D = 128
NPAD = 10240
NC = 2
NS = 16
EDGES_PER_SC = E // NC
EDGES_PER_TILE = EDGES_PER_SC // NS
BATCH = 80
NBATCH = EDGES_PER_TILE // BATCH
ROWS_PER_TILE = NPAD // NS
FLUSH_ROWS = 128
PRESERVE_C = 0.1

_MESH = dict(core_axis_name="c", subcore_axis_name="s")


def _zero_vec_ref(ref, nwords):
    zeros = jnp.zeros((16,), jnp.float32)

    def body(i, _):
        ref[pl.ds(i * 16, 16)] = zeros
        return 0

    lax.fori_loop(0, nwords // 16, body, 0)


def _zero_mat_ref(ref, nrows, ncols):
    zeros = jnp.zeros((16,), jnp.float32)

    def body(r, _):
        row = ref.at[r]
        for v in range(ncols // 16):
            row[pl.ds(v * 16, 16)] = zeros
        return 0

    lax.fori_loop(0, nrows, body, 0)


def _deg_body(dst_hbm, ew_hbm, deg_out, idx_v, upd_v, stage_v, deg_sp, sem):
    c = lax.axis_index("c")
    s = lax.axis_index("s")
    row0 = s * ROWS_PER_TILE

    _zero_vec_ref(stage_v, ROWS_PER_TILE)
    pltpu.sync_copy(stage_v, deg_sp.at[pl.ds(row0, ROWS_PER_TILE)])
    plsc.subcore_barrier()

    base0 = c * EDGES_PER_SC + s * EDGES_PER_TILE

    def batch(j, _):
        base = base0 + j * BATCH
        pltpu.sync_copy(dst_hbm.at[pl.ds(base, BATCH)], idx_v)
        pltpu.sync_copy(ew_hbm.at[pl.ds(base, BATCH)], upd_v)
        pltpu.sync_copy(upd_v, deg_sp.at[idx_v], add=True)
        return 0

    lax.fori_loop(0, NBATCH, batch, 0)
    plsc.subcore_barrier()

    pltpu.sync_copy(deg_sp.at[pl.ds(row0, ROWS_PER_TILE)], stage_v)
    pltpu.sync_copy(stage_v, deg_out.at[c, pl.ds(row0, ROWS_PER_TILE)])


@jax.jit
def _deg_call(dst, ew):
    k = pl.kernel(
        _deg_body,
        out_type=jax.ShapeDtypeStruct((NC, NPAD), jnp.float32),
        mesh=plsc.VectorSubcoreMesh(**_MESH),
        scratch_types=[
            pltpu.VMEM((BATCH,), jnp.int32),
            pltpu.VMEM((BATCH,), jnp.float32),
            pltpu.VMEM((ROWS_PER_TILE,), jnp.float32),
            pltpu.VMEM_SHARED((NPAD,), jnp.float32),
            pltpu.SemaphoreType.DMA,
        ],
    )
    return k(dst, ew)


def _agg_body(y_hbm, src_hbm, dst_hbm, ew_hbm, out_hbm,
              sidx, didx, ew_v, g_v, s_v, flush_v, acc_sp, sem):
    c = lax.axis_index("c")
    s = lax.axis_index("s")
    row0 = s * ROWS_PER_TILE

    _zero_mat_ref(flush_v, FLUSH_ROWS, D)
    for f in range(ROWS_PER_TILE // FLUSH_ROWS):
        pltpu.sync_copy(flush_v,
                        acc_sp.at[pl.ds(row0 + f * FLUSH_ROWS, FLUSH_ROWS)])
    plsc.subcore_barrier()

    base0 = c * EDGES_PER_SC + s * EDGES_PER_TILE

    def batch(j, _):
        base = base0 + j * BATCH
        pltpu.sync_copy(src_hbm.at[pl.ds(base, BATCH)], sidx)
        pltpu.sync_copy(dst_hbm.at[pl.ds(base, BATCH)], didx)
        pltpu.sync_copy(ew_hbm.at[pl.ds(base, BATCH)], ew_v)
        pltpu.async_copy(y_hbm.at[sidx], g_v, sem).wait()

        dnums = lax.GatherDimensionNumbers(
            offset_dims=(), collapsed_slice_dims=(0,), start_index_map=(0,))

        def group(g, _):
            ew16 = ew_v[pl.ds(g * 16, 16)]
            for l in range(16):
                bc = lax.gather(ew16, jnp.full((16, 1), l, jnp.int32),
                                dnums, slice_sizes=(1,),
                                mode=lax.GatherScatterMode.PROMISE_IN_BOUNDS)
                r = g * 16 + l
                gr = g_v.at[r]
                sr = s_v.at[r]
                for v in range(D // 16):
                    sr[pl.ds(v * 16, 16)] = gr[pl.ds(v * 16, 16)] * bc
            return 0

        lax.fori_loop(0, BATCH // 16, group, 0)
        pltpu.sync_copy(s_v, acc_sp.at[didx], add=True)
        return 0

    lax.fori_loop(0, NBATCH, batch, 0)
    plsc.subcore_barrier()

    for f in range(ROWS_PER_TILE // FLUSH_ROWS):
        r = row0 + f * FLUSH_ROWS
        pltpu.sync_copy(acc_sp.at[pl.ds(r, FLUSH_ROWS)], flush_v)
        pltpu.sync_copy(flush_v, out_hbm.at[c, pl.ds(r, FLUSH_ROWS)])


@jax.jit
def _agg_call(y, src, dst, ew):
    k = pl.kernel(
        _agg_body,
        out_type=jax.ShapeDtypeStruct((NC, NPAD, D), jnp.float32),
        mesh=plsc.VectorSubcoreMesh(**_MESH),
        scratch_types=[
            pltpu.VMEM((BATCH,), jnp.int32),
            pltpu.VMEM((BATCH,), jnp.int32),
            pltpu.VMEM((BATCH,), jnp.float32),
            pltpu.VMEM((BATCH, D), jnp.float32),
            pltpu.VMEM((BATCH, D), jnp.float32),
            pltpu.VMEM((FLUSH_ROWS, D), jnp.float32),
            pltpu.VMEM_SHARED((NPAD, D), jnp.float32),
            pltpu.SemaphoreType.DMA,
        ],
    )
    return k(y, src, dst, ew)


def _dinv_from(deg2):
    deg = deg2[0, :N] + deg2[1, :N] + 1.0
    safe = jnp.where(deg > 0.0, deg, 1.0)
    return jnp.where(deg > 0.0, lax.rsqrt(safe), 0.0)[:, None]


def _tc0_body(x_ref, w_ref, deg_ref, y_ref, dinv_ref):
    dinv = _dinv_from(deg_ref[...])
    xw = jnp.dot(x_ref[...], w_ref[...], preferred_element_type=jnp.float32)
    y_ref[...] = xw * dinv
    dinv_ref[...] = dinv


@jax.jit
def _tc0_call(x, W1, deg2):
    return pl.pallas_call(
        _tc0_body,
        out_shape=(
            jax.ShapeDtypeStruct((N, D), jnp.float32),
            jax.ShapeDtypeStruct((N, 1), jnp.float32),
        ),
    )(x, W1, deg2)


def _mix_body(agg_ref, y_ref, dinv_ref, b_ref, tp_ref, w_ref,
              temp_ref, ynext_ref):
    agg = agg_ref[0, :N, :] + agg_ref[1, :N, :]
    dinv = dinv_ref[...]
    out = dinv * (agg + y_ref[...]) + b_ref[...]
    temp = (1.0 - PRESERVE_C) * out + PRESERVE_C * tp_ref[...]
    temp_ref[...] = temp
    ynext_ref[...] = jnp.dot(temp, w_ref[...],
                             preferred_element_type=jnp.float32) * dinv


@jax.jit
def _mix_call(agg2, y, dinv, b, temp_prev, Wn):
    return pl.pallas_call(
        _mix_body,
        out_shape=(
            jax.ShapeDtypeStruct((N, D), jnp.float32),
            jax.ShapeDtypeStruct((N, D), jnp.float32),
        ),
    )(agg2, y, dinv, b, temp_prev, Wn)


def _fin_body(agg_ref, y_ref, dinv_ref, b_ref, tp_ref, temp_ref):
    agg = agg_ref[0, :N, :] + agg_ref[1, :N, :]
    out = dinv_ref[...] * (agg + y_ref[...]) + b_ref[...]
    temp_ref[...] = (1.0 - PRESERVE_C) * out + PRESERVE_C * tp_ref[...]


@jax.jit
def _fin_call(agg2, y, dinv, b, temp_prev):
    return pl.pallas_call(
        _fin_body,
        out_shape=jax.ShapeDtypeStruct((N, D), jnp.float32),
    )(agg2, y, dinv, b, temp_prev)


def kernel(skill_embed, adj_list, edge_attr, W1, b1, W2, b2, W3, b3):
    src = adj_list[0]
    dst = adj_list[1]

    deg2 = _deg_call(dst, edge_attr)
    y1, dinv = _tc0_call(skill_embed, W1, deg2)

    agg1 = _agg_call(y1, src, dst, edge_attr)
    temp1, y2 = _mix_call(agg1, y1, dinv, b1.reshape(1, D), skill_embed, W2)

    agg2 = _agg_call(y2, src, dst, edge_attr)
    temp2, y3 = _mix_call(agg2, y2, dinv, b2.reshape(1, D), temp1, W3)

    agg3 = _agg_call(y3, src, dst, edge_attr)
    return _fin_call(agg3, y3, dinv, b3.reshape(1, D), temp2)


# speedup vs baseline: 21.0073x; 2.6607x over previous
import functools

import jax
import jax.numpy as jnp
from jax import lax
from jax.experimental import pallas as pl
from jax.experimental.pallas import tpu as pltpu
from jax.experimental.pallas import tpu_sc as plsc

N = 10000
E = 320000
D = 128
NPAD = 10240
NC = 2
NS = 16
EDGES_PER_SC = E // NC
EDGES_PER_TILE = EDGES_PER_SC // NS
BATCH = 80
NBATCH = EDGES_PER_TILE // BATCH
ROWS_PER_TILE = NPAD // NS
FLUSH_ROWS = 128
PRESERVE_C = 0.1

_MESH = dict(core_axis_name="c", subcore_axis_name="s")


def _zero_vec_ref(ref, nwords):
    zeros = jnp.zeros((16,), jnp.float32)

    def body(i, _):
        ref[pl.ds(i * 16, 16)] = zeros
        return 0

    lax.fori_loop(0, nwords // 16, body, 0)


def _zero_mat_ref(ref, nrows, ncols):
    zeros = jnp.zeros((16,), jnp.float32)

    def body(r, _):
        row = ref.at[r]
        for v in range(ncols // 16):
            row[pl.ds(v * 16, 16)] = zeros
        return 0

    lax.fori_loop(0, nrows, body, 0)


def _deg_body(dst_hbm, ew_hbm, deg_out, idx_v, upd_v, stage_v, deg_sp, sem):
    c = lax.axis_index("c")
    s = lax.axis_index("s")
    row0 = s * ROWS_PER_TILE

    _zero_vec_ref(stage_v, ROWS_PER_TILE)
    pltpu.sync_copy(stage_v, deg_sp.at[pl.ds(row0, ROWS_PER_TILE)])

    t0 = c * NS + s
    pltpu.sync_copy(dst_hbm.at[t0], idx_v)
    pltpu.sync_copy(ew_hbm.at[t0], upd_v)
    plsc.subcore_barrier()

    def batch(j, _):
        pltpu.sync_copy(upd_v.at[j], deg_sp.at[idx_v.at[j]], add=True)
        return 0

    lax.fori_loop(0, NBATCH, batch, 0)
    plsc.subcore_barrier()

    pltpu.sync_copy(deg_sp.at[pl.ds(row0, ROWS_PER_TILE)], stage_v)
    pltpu.sync_copy(stage_v, deg_out.at[c, pl.ds(row0, ROWS_PER_TILE)])


@jax.jit
def _deg_call(dst2, ew2):
    k = pl.kernel(
        _deg_body,
        out_type=jax.ShapeDtypeStruct((NC, NPAD), jnp.float32),
        mesh=plsc.VectorSubcoreMesh(**_MESH),
        scratch_types=[
            pltpu.VMEM((NBATCH, BATCH), jnp.int32),
            pltpu.VMEM((NBATCH, BATCH), jnp.float32),
            pltpu.VMEM((ROWS_PER_TILE,), jnp.float32),
            pltpu.VMEM_SHARED((NPAD,), jnp.float32),
            pltpu.SemaphoreType.DMA,
        ],
    )
    return k(dst2, ew2)


_DNUMS = lax.GatherDimensionNumbers(
    offset_dims=(), collapsed_slice_dims=(0,), start_index_map=(0,))


def _agg_body(y_hbm, src_hbm, dst_hbm, ew_hbm, out_hbm,
              sidx, ew0, ew1, didx0, didx1, g0, g1, acc_sp,
              gsem0, gsem1, isem0, isem1, esem0, esem1):
    c = lax.axis_index("c")
    s = lax.axis_index("s")
    row0 = s * ROWS_PER_TILE

    _zero_mat_ref(g0, BATCH, D)
    for f in range(ROWS_PER_TILE // BATCH):
        pltpu.sync_copy(g0, acc_sp.at[pl.ds(row0 + f * BATCH, BATCH)])

    t0 = c * NS + s
    pltpu.sync_copy(src_hbm.at[t0], sidx)
    plsc.subcore_barrier()

    def start_gather(j, gbuf, gsem):
        pltpu.make_async_copy(y_hbm.at[sidx.at[j]], gbuf, gsem).start()

    def wait_gather(j, gbuf, gsem):
        pltpu.make_async_copy(y_hbm.at[sidx.at[j]], gbuf, gsem).wait()

    ebase = t0 * EDGES_PER_TILE

    def start_didx(j, dbuf, isem):
        pltpu.make_async_copy(dst_hbm.at[pl.ds(ebase + j * BATCH, BATCH)],
                              dbuf, isem).start()

    def wait_didx(j, dbuf, isem):
        pltpu.make_async_copy(dst_hbm.at[pl.ds(ebase + j * BATCH, BATCH)],
                              dbuf, isem).wait()

    def start_ew(j, ebuf, esem):
        pltpu.make_async_copy(ew_hbm.at[pl.ds(ebase + j * BATCH, BATCH)],
                              ebuf, esem).start()

    def wait_ew(j, ebuf, esem):
        pltpu.make_async_copy(ew_hbm.at[pl.ds(ebase + j * BATCH, BATCH)],
                              ebuf, esem).wait()

    def scale_scatter(j, gbuf, dbuf, isem, ebuf, esem):
        wait_ew(j, ebuf, esem)

        def group(g, _):
            ew16 = ebuf[pl.ds(g * 16, 16)]
            for l in range(16):
                bc = lax.gather(ew16, jnp.full((16, 1), l, jnp.int32),
                                _DNUMS, slice_sizes=(1,),
                                mode=lax.GatherScatterMode.PROMISE_IN_BOUNDS)
                gr = gbuf.at[g * 16 + l]
                for v in range(D // 16):
                    gr[pl.ds(v * 16, 16)] = gr[pl.ds(v * 16, 16)] * bc
            return 0

        lax.fori_loop(0, BATCH // 16, group, 0)
        wait_didx(j, dbuf, isem)
        pltpu.sync_copy(gbuf, acc_sp.at[dbuf], add=True)

    start_didx(0, didx0, isem0)
    start_ew(0, ew0, esem0)
    start_gather(0, g0, gsem0)

    def pair(t, _):
        j = 2 * t
        wait_gather(j, g0, gsem0)
        start_gather(j + 1, g1, gsem1)
        start_didx(j + 1, didx1, isem1)
        start_ew(j + 1, ew1, esem1)
        scale_scatter(j, g0, didx0, isem0, ew0, esem0)
        wait_gather(j + 1, g1, gsem1)
        start_gather(j + 2, g0, gsem0)
        start_didx(j + 2, didx0, isem0)
        start_ew(j + 2, ew0, esem0)
        scale_scatter(j + 1, g1, didx1, isem1, ew1, esem1)
        return 0

    lax.fori_loop(0, (NBATCH - 1) // 2, pair, 0)
    wait_gather(NBATCH - 1, g0, gsem0)
    scale_scatter(NBATCH - 1, g0, didx0, isem0, ew0, esem0)

    plsc.subcore_barrier()

    for f in range(ROWS_PER_TILE // BATCH):
        r = row0 + f * BATCH
        pltpu.sync_copy(acc_sp.at[pl.ds(r, BATCH)], g0)
        pltpu.sync_copy(g0, out_hbm.at[c, pl.ds(r, BATCH)])


@jax.jit
def _agg_call(y, src2, dst2, ew2):
    k = pl.kernel(
        _agg_body,
        out_type=jax.ShapeDtypeStruct((NC, NPAD, D), jnp.float32),
        mesh=plsc.VectorSubcoreMesh(**_MESH),
        scratch_types=[
            pltpu.VMEM((NBATCH, BATCH), jnp.int32),
            pltpu.VMEM((BATCH,), jnp.float32),
            pltpu.VMEM((BATCH,), jnp.float32),
            pltpu.VMEM((BATCH,), jnp.int32),
            pltpu.VMEM((BATCH,), jnp.int32),
            pltpu.VMEM((BATCH, D), jnp.float32),
            pltpu.VMEM((BATCH, D), jnp.float32),
            pltpu.VMEM_SHARED((NPAD, D), jnp.float32),
            pltpu.SemaphoreType.DMA,
            pltpu.SemaphoreType.DMA,
            pltpu.SemaphoreType.DMA,
            pltpu.SemaphoreType.DMA,
            pltpu.SemaphoreType.DMA,
            pltpu.SemaphoreType.DMA,
        ],
    )
    return k(y, src2, dst2, ew2)


def _dinv_from(deg2):
    deg = deg2[0, :N] + deg2[1, :N] + 1.0
    safe = jnp.where(deg > 0.0, deg, 1.0)
    return jnp.where(deg > 0.0, lax.rsqrt(safe), 0.0)[:, None]


def _tc0_body(x_ref, w_ref, deg_ref, y_ref, dinv_ref):
    dinv = _dinv_from(deg_ref[...])
    xw = jnp.dot(x_ref[...], w_ref[...], preferred_element_type=jnp.float32)
    y_ref[...] = xw * dinv
    dinv_ref[...] = dinv


@jax.jit
def _tc0_call(x, W1, deg2):
    return pl.pallas_call(
        _tc0_body,
        out_shape=(
            jax.ShapeDtypeStruct((N, D), jnp.float32),
            jax.ShapeDtypeStruct((N, 1), jnp.float32),
        ),
    )(x, W1, deg2)


def _mix_body(agg_ref, y_ref, dinv_ref, b_ref, tp_ref, w_ref,
              temp_ref, ynext_ref):
    agg = agg_ref[0, :N, :] + agg_ref[1, :N, :]
    dinv = dinv_ref[...]
    out = dinv * (agg + y_ref[...]) + b_ref[...]
    temp = (1.0 - PRESERVE_C) * out + PRESERVE_C * tp_ref[...]
    temp_ref[...] = temp
    ynext_ref[...] = jnp.dot(temp, w_ref[...],
                             preferred_element_type=jnp.float32) * dinv


@jax.jit
def _mix_call(agg2, y, dinv, b, temp_prev, Wn):
    return pl.pallas_call(
        _mix_body,
        out_shape=(
            jax.ShapeDtypeStruct((N, D), jnp.float32),
            jax.ShapeDtypeStruct((N, D), jnp.float32),
        ),
    )(agg2, y, dinv, b, temp_prev, Wn)


def _fin_body(agg_ref, y_ref, dinv_ref, b_ref, tp_ref, temp_ref):
    agg = agg_ref[0, :N, :] + agg_ref[1, :N, :]
    out = dinv_ref[...] * (agg + y_ref[...]) + b_ref[...]
    temp_ref[...] = (1.0 - PRESERVE_C) * out + PRESERVE_C * tp_ref[...]


@jax.jit
def _fin_call(agg2, y, dinv, b, temp_prev):
    return pl.pallas_call(
        _fin_body,
        out_shape=jax.ShapeDtypeStruct((N, D), jnp.float32),
    )(agg2, y, dinv, b, temp_prev)


def kernel(skill_embed, adj_list, edge_attr, W1, b1, W2, b2, W3, b3):
    src = adj_list[0].reshape(NC * NS, NBATCH, BATCH)
    dst = adj_list[1]
    dst3 = adj_list[1].reshape(NC * NS, NBATCH, BATCH)
    ew3 = edge_attr.reshape(NC * NS, NBATCH, BATCH)

    deg2 = _deg_call(dst3, ew3)
    y1, dinv = _tc0_call(skill_embed, W1, deg2)

    agg1 = _agg_call(y1, src, dst, edge_attr)
    temp1, y2 = _mix_call(agg1, y1, dinv, b1.reshape(1, D), skill_embed, W2)

    agg2 = _agg_call(y2, src, dst, edge_attr)
    temp2, y3 = _mix_call(agg2, y2, dinv, b2.reshape(1, D), temp1, W3)

    agg3 = _agg_call(y3, src, dst, edge_attr)
    return _fin_call(agg3, y3, dinv, b3.reshape(1, D), temp2)
